# Initial kernel scaffold; baseline (speedup 1.0000x reference)
import jax
import jax.numpy as jnp
from jax.experimental import pallas as pl


def kernel(x, edge_index, edge_attr, pos, v, params):
    raise NotImplementedError("write your pallas kernel here")


# speedup vs baseline: 2.9233x; 2.9233x over previous
import functools

import jax
import jax.numpy as jnp
from jax import lax
from jax.experimental import pallas as pl
from jax.experimental.pallas import tpu as pltpu
from jax.experimental.pallas import tpu_sc as plsc

_N = 10000
_E = 320000
_H = 128
_L = 3

_NC = 2
_NS = 16
_NW = _NC * _NS

_CHUNK = 80
_CPW = 125
_EPW = _CPW * _CHUNK
_RPT = 624

_BE = 2000
_GE = _E // _BE

@functools.cache
def _mesh():
    return plsc.VectorSubcoreMesh(
        core_axis_name="c", subcore_axis_name="s", num_cores=_NC, num_subcores=_NS
    )


def _sc_gather(table, idx3):
    def body(table_ref, idx_ref, out_ref, idx_v, rows_v, sem):
        wid = lax.axis_index("s") * _NC + lax.axis_index("c")
        pltpu.sync_copy(idx_ref.at[wid], idx_v)
        base = wid * _EPW

        def step(j, carry):
            pltpu.async_copy(table_ref.at[idx_v.at[j]], rows_v, sem).wait()
            pltpu.sync_copy(rows_v, out_ref.at[pl.ds(base + j * _CHUNK, _CHUNK)])
            return carry

        lax.fori_loop(0, _CPW, step, 0)

    return pl.kernel(
        body,
        out_type=jax.ShapeDtypeStruct((_E, _H), jnp.float32),
        mesh=_mesh(),
        scratch_types=[
            pltpu.VMEM((_CPW, _CHUNK), jnp.int32),
            pltpu.VMEM((_CHUNK, _H), jnp.float32),
            pltpu.SemaphoreType.DMA,
        ],
    )(table, idx3)


def _sc_scatter_add(vals, idx3, zeros, hc):
    def body(vals_ref, idx_ref, zeros_ref, out_ref, idx_v, rows_v, acc, sem):
        cid = lax.axis_index("c")
        sid = lax.axis_index("s")
        wid = sid * _NC + cid
        @pl.when(sid < _NS - 1)
        def _():
            pltpu.sync_copy(
                zeros_ref.at[pl.ds(sid * _RPT, _RPT)],
                acc.at[pl.ds(sid * _RPT, _RPT)],
            )

        @pl.when(sid == _NS - 1)
        def _():
            lastn = _N - (_NS - 1) * _RPT
            pltpu.sync_copy(
                zeros_ref.at[pl.ds((_NS - 1) * _RPT, lastn)],
                acc.at[pl.ds((_NS - 1) * _RPT, lastn)],
            )

        pltpu.sync_copy(idx_ref.at[wid], idx_v)
        plsc.subcore_barrier()
        base = wid * _EPW

        def step(j, carry):
            pltpu.sync_copy(
                vals_ref.at[pl.ds(base + j * _CHUNK, _CHUNK)], rows_v
            )
            pltpu.sync_copy(rows_v, acc.at[idx_v.at[j]], add=True)
            return carry

        lax.fori_loop(0, _CPW, step, 0)
        plsc.subcore_barrier()

        @pl.when(sid < _NS - 1)
        def _():
            pltpu.sync_copy(
                acc.at[pl.ds(sid * _RPT, _RPT)],
                out_ref.at[cid, pl.ds(sid * _RPT, _RPT)],
            )

        @pl.when(sid == _NS - 1)
        def _():
            lastn = _N - (_NS - 1) * _RPT
            pltpu.sync_copy(
                acc.at[pl.ds((_NS - 1) * _RPT, lastn)],
                out_ref.at[cid, pl.ds((_NS - 1) * _RPT, lastn)],
            )

    return pl.kernel(
        body,
        out_type=jax.ShapeDtypeStruct((_NC, _N, hc), jnp.float32),
        mesh=_mesh(),
        scratch_types=[
            pltpu.VMEM((_CPW, _CHUNK), jnp.int32),
            pltpu.VMEM((_CHUNK, hc), jnp.float32),
            pltpu.VMEM_SHARED((_N, hc), jnp.float32),
            pltpu.SemaphoreType.DMA,
        ],
    )(vals, idx3, zeros)


def _gelu(x):
    return 0.5 * x * (1.0 + lax.erf(x * 0.7071067811865476))


def _ln(h, g, b):
    m = jnp.mean(h, axis=-1, keepdims=True)
    d = h - m
    v = jnp.mean(d * d, axis=-1, keepdims=True)
    return d * lax.rsqrt(v + 1e-5) * g + b


def _full(shape):
    return pl.BlockSpec(shape, lambda i: (0,) * len(shape))


def _proj_body(x_ref, w1a_ref, w1b_ref, c_ref, xa_ref, xb_ref):
    x = x_ref[...]
    xa_ref[...] = (
        jnp.dot(x, w1a_ref[...], preferred_element_type=jnp.float32) + c_ref[0:1, :]
    )
    xb_ref[...] = jnp.dot(x, w1b_ref[...], preferred_element_type=jnp.float32)


def _tc_proj(x, w1a, w1b, consts):
    return pl.pallas_call(
        _proj_body,
        out_shape=(
            jax.ShapeDtypeStruct((_N, _H), jnp.float32),
            jax.ShapeDtypeStruct((_N, _H), jnp.float32),
        ),
    )(x, w1a, w1b, consts)


def _msg_body(ga_ref, gb_ref, ea_ref, w1c_ref, w2_ref, c_ref, out_ref):
    h = (
        ga_ref[...]
        + gb_ref[...]
        + jnp.dot(ea_ref[...], w1c_ref[...], preferred_element_type=jnp.float32)
    )
    h = jnp.dot(_gelu(h), w2_ref[...], preferred_element_type=jnp.float32)
    h = h + c_ref[0:1, :]
    out_ref[...] = _ln(h, c_ref[1:2, :], c_ref[2:3, :])


def _tc_msg(ga, gb, ea, w1c, w2, consts):
    eb = pl.BlockSpec((_BE, _H), lambda i: (i, 0))
    return pl.pallas_call(
        _msg_body,
        grid=(_GE,),
        in_specs=[eb, eb, eb, _full((_H, _H)), _full((_H, _H)), _full((8, _H))],
        out_specs=eb,
        out_shape=jax.ShapeDtypeStruct((_E, _H), jnp.float32),
    )(ga, gb, ea, w1c, w2, consts)


def _flux_body(en_ref, ei_ref, ea_ref, w1_ref, w2_ref, c_ref, fl_ref, eo_ref):
    en = en_ref[...]
    ei = ei_ref[...]
    h = jnp.dot(en + ei, w1_ref[...], preferred_element_type=jnp.float32)
    h = h + c_ref[0:1, :]
    h = jnp.dot(_gelu(h), w2_ref[...], preferred_element_type=jnp.float32)
    h = h + c_ref[1:2, :]
    fh = _ln(h, c_ref[2:3, :], c_ref[3:4, :])
    fl = en - ei + fh
    fl_ref[...] = fl
    eo_ref[...] = fl + ea_ref[...]


def _tc_flux(en, ei, ea, w1, w2, consts):
    eb = pl.BlockSpec((_BE, _H), lambda i: (i, 0))
    return pl.pallas_call(
        _flux_body,
        grid=(_GE,),
        in_specs=[eb, eb, eb, _full((_H, _H)), _full((_H, _H)), _full((8, _H))],
        out_specs=(eb, eb),
        out_shape=(
            jax.ShapeDtypeStruct((_E, _H), jnp.float32),
            jax.ShapeDtypeStruct((_E, _H), jnp.float32),
        ),
    )(en, ei, ea, w1, w2, consts)


def _upd_body(x_ref, aggp_ref, cntp_ref, w1a_ref, w1b_ref, w2_ref, c_ref, out_ref):
    x = x_ref[...]
    a = aggp_ref[0] + aggp_ref[1]
    cnt = jnp.maximum(cntp_ref[0, :, 0:1] + cntp_ref[1, :, 0:1], 1.0)
    agg = a / cnt
    h = jnp.dot(x, w1a_ref[...], preferred_element_type=jnp.float32) + jnp.dot(
        agg, w1b_ref[...], preferred_element_type=jnp.float32
    )
    h = h + c_ref[0:1, :]
    h = jnp.dot(_gelu(h), w2_ref[...], preferred_element_type=jnp.float32)
    h = h + c_ref[1:2, :]
    out_ref[...] = x + _ln(h, c_ref[2:3, :], c_ref[3:4, :])


def _tc_upd(x, aggp, cntp, w1a, w1b, w2, consts):
    return pl.pallas_call(
        _upd_body,
        out_shape=jax.ShapeDtypeStruct((_N, _H), jnp.float32),
    )(x, aggp, cntp, w1a, w1b, w2, consts)


def kernel(x, edge_index, edge_attr, pos, v, params):
    row = edge_index[0]
    col = edge_index[1]
    inv_perm = jnp.argsort(col * _N + row).astype(jnp.int32)

    row2 = row.reshape(_NW, _CPW, _CHUNK)
    col2 = col.reshape(_NW, _CPW, _CHUNK)
    perm2 = inv_perm.reshape(_NW, _CPW, _CHUNK)

    zeros_h = jnp.zeros((_N, _H), jnp.float32)
    ones_e = jnp.ones((_E, _H), jnp.float32)

    cntp = _sc_scatter_add(ones_e, row2, zeros_h, _H)[:, :, :16]

    p = params
    for l in range(_L):
        w1 = p["msg_w1"][l]
        msg_c = jnp.stack(
            [p["msg_b2"][l], p["ln_g"][l], p["ln_b"][l]] + [p["ln_b"][l]] * 5
        )
        proj_c = jnp.stack([p["msg_b1"][l]] * 8)
        flux_c = jnp.stack(
            [p["flux_b1"][l], p["flux_b2"][l], p["ln_g"][l], p["ln_b"][l]] * 2
        )
        upd_c = jnp.stack(
            [p["upd_b1"][l], p["upd_b2"][l], p["ln_g"][l], p["ln_b"][l]] * 2
        )

        xa, xb = _tc_proj(x, w1[:_H], w1[_H : 2 * _H], proj_c)
        ga = _sc_gather(xa, row2)
        gb = _sc_gather(xb, col2)
        ea_new = _tc_msg(ga, gb, edge_attr, w1[2 * _H :], p["msg_w2"][l], msg_c)
        ea_inv = _sc_gather(ea_new, perm2)
        flux, edge_attr = _tc_flux(
            ea_new, ea_inv, edge_attr, p["flux_w1"][l], p["flux_w2"][l], flux_c
        )
        aggp = _sc_scatter_add(flux, row2, zeros_h, _H)
        x = _tc_upd(
            x,
            aggp,
            cntp,
            p["upd_w1"][l][:_H],
            p["upd_w1"][l][_H:],
            p["upd_w2"][l],
            upd_c,
        )
    return x, edge_attr


# speedup vs baseline: 3.3247x; 1.1373x over previous
import functools

import jax
import jax.numpy as jnp
from jax import lax
from jax.experimental import pallas as pl
from jax.experimental.pallas import tpu as pltpu
from jax.experimental.pallas import tpu_sc as plsc

_N = 10000
_E = 320000
_H = 128
_L = 3

_NC = 2
_NS = 16
_NW = _NC * _NS

_CHUNK = 80
_CPW = 125
_EPW = _CPW * _CHUNK
_RPT = 624

_BE = 2000
_GE = _E // _BE

@functools.cache
def _mesh():
    return plsc.VectorSubcoreMesh(
        core_axis_name="c", subcore_axis_name="s", num_cores=_NC, num_subcores=_NS
    )


def _pipe2(n, mk_in, mk_out, g0, g1, s0, s1, out_add=False):
    def step(j, carry):
        @pl.when(j % 2 == 0)
        def _():
            mk_in(j, 0, g0).wait()

            @pl.when(j + 1 < n)
            def _():
                @pl.when(j >= 1)
                def _():
                    mk_out(j - 1, 1, s1).wait()

                mk_in(j + 1, 1, g1).start()

            mk_out(j, 0, s0).start(add=out_add)

        @pl.when(j % 2 == 1)
        def _():
            mk_in(j, 1, g1).wait()

            @pl.when(j + 1 < n)
            def _():
                mk_out(j - 1, 0, s0).wait()
                mk_in(j + 1, 0, g0).start()

            mk_out(j, 1, s1).start(add=out_add)

        return carry

    mk_in(0, 0, g0).start()
    lax.fori_loop(0, n, step, 0)
    mk_out(n - 2, (n - 2) % 2, s0 if (n - 2) % 2 == 0 else s1).wait()
    mk_out(n - 1, (n - 1) % 2, s0 if (n - 1) % 2 == 0 else s1).wait()


def _sc_gather2(ta, tb, idxa3, idxb3):
    def body(ta_ref, tb_ref, ia_ref, ib_ref, oa_ref, ob_ref,
             ia_v, ib_v, rows_v, g0, g1, s0, s1):
        wid = lax.axis_index("s") * _NC + lax.axis_index("c")
        pltpu.sync_copy(ia_ref.at[wid], ia_v)
        pltpu.sync_copy(ib_ref.at[wid], ib_v)
        base = wid * _EPW

        def mk_in(j, b, sem):
            half = j // 2
            if b == 0:
                return pltpu.make_async_copy(
                    ta_ref.at[ia_v.at[half]], rows_v.at[b], sem
                )
            return pltpu.make_async_copy(
                tb_ref.at[ib_v.at[half]], rows_v.at[b], sem
            )

        def mk_out(j, b, sem):
            half = j // 2
            dst = oa_ref if b == 0 else ob_ref
            return pltpu.make_async_copy(
                rows_v.at[b], dst.at[pl.ds(base + half * _CHUNK, _CHUNK)], sem
            )

        _pipe2(2 * _CPW, mk_in, mk_out, g0, g1, s0, s1)

    return pl.kernel(
        body,
        out_type=(
            jax.ShapeDtypeStruct((_E, _H), jnp.float32),
            jax.ShapeDtypeStruct((_E, _H), jnp.float32),
        ),
        mesh=_mesh(),
        scratch_types=[
            pltpu.VMEM((_CPW, _CHUNK), jnp.int32),
            pltpu.VMEM((_CPW, _CHUNK), jnp.int32),
            pltpu.VMEM((2, _CHUNK, _H), jnp.float32),
            pltpu.SemaphoreType.DMA,
            pltpu.SemaphoreType.DMA,
            pltpu.SemaphoreType.DMA,
            pltpu.SemaphoreType.DMA,
        ],
    )(ta, tb, idxa3, idxb3)


def _sc_gather(table, idx3):
    def body(table_ref, idx_ref, out_ref, idx_v, rows_v, g0, g1, s0, s1):
        wid = lax.axis_index("s") * _NC + lax.axis_index("c")
        pltpu.sync_copy(idx_ref.at[wid], idx_v)
        base = wid * _EPW

        def mk_in(j, b, sem):
            return pltpu.make_async_copy(table_ref.at[idx_v.at[j]], rows_v.at[b], sem)

        def mk_out(j, b, sem):
            return pltpu.make_async_copy(
                rows_v.at[b], out_ref.at[pl.ds(base + j * _CHUNK, _CHUNK)], sem
            )

        _pipe2(_CPW, mk_in, mk_out, g0, g1, s0, s1)

    return pl.kernel(
        body,
        out_type=jax.ShapeDtypeStruct((_E, _H), jnp.float32),
        mesh=_mesh(),
        scratch_types=[
            pltpu.VMEM((_CPW, _CHUNK), jnp.int32),
            pltpu.VMEM((2, _CHUNK, _H), jnp.float32),
            pltpu.SemaphoreType.DMA,
            pltpu.SemaphoreType.DMA,
            pltpu.SemaphoreType.DMA,
            pltpu.SemaphoreType.DMA,
        ],
    )(table, idx3)


def _sc_scatter_add(vals, idx3, zeros, hc):
    def body(vals_ref, idx_ref, zeros_ref, out_ref, idx_v, rows_v, acc,
             g0, g1, s0, s1):
        cid = lax.axis_index("c")
        sid = lax.axis_index("s")
        wid = sid * _NC + cid
        @pl.when(sid < _NS - 1)
        def _():
            pltpu.sync_copy(
                zeros_ref.at[pl.ds(sid * _RPT, _RPT)],
                acc.at[pl.ds(sid * _RPT, _RPT)],
            )

        @pl.when(sid == _NS - 1)
        def _():
            lastn = _N - (_NS - 1) * _RPT
            pltpu.sync_copy(
                zeros_ref.at[pl.ds((_NS - 1) * _RPT, lastn)],
                acc.at[pl.ds((_NS - 1) * _RPT, lastn)],
            )

        pltpu.sync_copy(idx_ref.at[wid], idx_v)
        plsc.subcore_barrier()
        base = wid * _EPW

        def mk_in(j, b, sem):
            return pltpu.make_async_copy(
                vals_ref.at[pl.ds(base + j * _CHUNK, _CHUNK)], rows_v.at[b], sem
            )

        def mk_out(j, b, sem):
            return pltpu.make_async_copy(rows_v.at[b], acc.at[idx_v.at[j]], sem)

        _pipe2(_CPW, mk_in, mk_out, g0, g1, s0, s1, out_add=True)
        plsc.subcore_barrier()

        @pl.when(sid < _NS - 1)
        def _():
            pltpu.sync_copy(
                acc.at[pl.ds(sid * _RPT, _RPT)],
                out_ref.at[cid, pl.ds(sid * _RPT, _RPT)],
            )

        @pl.when(sid == _NS - 1)
        def _():
            lastn = _N - (_NS - 1) * _RPT
            pltpu.sync_copy(
                acc.at[pl.ds((_NS - 1) * _RPT, lastn)],
                out_ref.at[cid, pl.ds((_NS - 1) * _RPT, lastn)],
            )

    return pl.kernel(
        body,
        out_type=jax.ShapeDtypeStruct((_NC, _N, hc), jnp.float32),
        mesh=_mesh(),
        scratch_types=[
            pltpu.VMEM((_CPW, _CHUNK), jnp.int32),
            pltpu.VMEM((2, _CHUNK, hc), jnp.float32),
            pltpu.VMEM_SHARED((_N, hc), jnp.float32),
            pltpu.SemaphoreType.DMA,
            pltpu.SemaphoreType.DMA,
            pltpu.SemaphoreType.DMA,
            pltpu.SemaphoreType.DMA,
        ],
    )(vals, idx3, zeros)


def _sc_count(idx3, zeros):
    def body(idx_ref, zeros_ref, out_ref, idx_v, ones_v, acc, sem):
        cid = lax.axis_index("c")
        sid = lax.axis_index("s")
        wid = sid * _NC + cid

        @pl.when(sid < _NS - 1)
        def _():
            pltpu.sync_copy(
                zeros_ref.at[pl.ds(sid * _RPT, _RPT)],
                acc.at[pl.ds(sid * _RPT, _RPT)],
            )

        @pl.when(sid == _NS - 1)
        def _():
            lastn = _N - (_NS - 1) * _RPT
            pltpu.sync_copy(
                zeros_ref.at[pl.ds((_NS - 1) * _RPT, lastn)],
                acc.at[pl.ds((_NS - 1) * _RPT, lastn)],
            )

        pltpu.sync_copy(idx_ref.at[wid], idx_v)

        def fill(i, c):
            def fill2(k, c2):
                ones_v[i, pl.ds(k * 16, 16)] = jnp.ones((16,), jnp.float32)
                return c2

            return lax.fori_loop(0, _H // 16, fill2, c)

        lax.fori_loop(0, _CHUNK, fill, 0)
        plsc.subcore_barrier()

        def mk(j):
            return pltpu.make_async_copy(ones_v, acc.at[idx_v.at[j]], sem)

        def step(j, carry):
            mk(j).start(add=True)

            @pl.when(j >= 4)
            def _():
                mk(j - 4).wait()

            return carry

        lax.fori_loop(0, _CPW, step, 0)
        for j in range(4):
            mk(_CPW - 4 + j).wait()
        plsc.subcore_barrier()

        @pl.when(sid < _NS - 1)
        def _():
            pltpu.sync_copy(
                acc.at[pl.ds(sid * _RPT, _RPT)],
                out_ref.at[cid, pl.ds(sid * _RPT, _RPT)],
            )

        @pl.when(sid == _NS - 1)
        def _():
            lastn = _N - (_NS - 1) * _RPT
            pltpu.sync_copy(
                acc.at[pl.ds((_NS - 1) * _RPT, lastn)],
                out_ref.at[cid, pl.ds((_NS - 1) * _RPT, lastn)],
            )

    return pl.kernel(
        body,
        out_type=jax.ShapeDtypeStruct((_NC, _N, _H), jnp.float32),
        mesh=_mesh(),
        scratch_types=[
            pltpu.VMEM((_CPW, _CHUNK), jnp.int32),
            pltpu.VMEM((_CHUNK, _H), jnp.float32),
            pltpu.VMEM_SHARED((_N, _H), jnp.float32),
            pltpu.SemaphoreType.DMA,
        ],
    )(idx3, zeros)


def _gelu(x):
    return 0.5 * x * (1.0 + lax.erf(x * 0.7071067811865476))


def _ln(h, g, b):
    m = jnp.mean(h, axis=-1, keepdims=True)
    d = h - m
    v = jnp.mean(d * d, axis=-1, keepdims=True)
    return d * lax.rsqrt(v + 1e-5) * g + b


def _full(shape):
    return pl.BlockSpec(shape, lambda i: (0,) * len(shape))


def _proj_body(x_ref, w1a_ref, w1b_ref, c_ref, xa_ref, xb_ref):
    x = x_ref[...]
    xa_ref[...] = (
        jnp.dot(x, w1a_ref[...], preferred_element_type=jnp.float32) + c_ref[0:1, :]
    )
    xb_ref[...] = jnp.dot(x, w1b_ref[...], preferred_element_type=jnp.float32)


def _tc_proj(x, w1a, w1b, consts):
    return pl.pallas_call(
        _proj_body,
        out_shape=(
            jax.ShapeDtypeStruct((_N, _H), jnp.float32),
            jax.ShapeDtypeStruct((_N, _H), jnp.float32),
        ),
    )(x, w1a, w1b, consts)


def _msg_body(ga_ref, gb_ref, ea_ref, w1c_ref, w2_ref, c_ref, out_ref):
    h = (
        ga_ref[...]
        + gb_ref[...]
        + jnp.dot(ea_ref[...], w1c_ref[...], preferred_element_type=jnp.float32)
    )
    h = jnp.dot(_gelu(h), w2_ref[...], preferred_element_type=jnp.float32)
    h = h + c_ref[0:1, :]
    out_ref[...] = _ln(h, c_ref[1:2, :], c_ref[2:3, :])


def _tc_msg(ga, gb, ea, w1c, w2, consts):
    eb = pl.BlockSpec((_BE, _H), lambda i: (i, 0))
    return pl.pallas_call(
        _msg_body,
        grid=(_GE,),
        in_specs=[eb, eb, eb, _full((_H, _H)), _full((_H, _H)), _full((8, _H))],
        out_specs=eb,
        out_shape=jax.ShapeDtypeStruct((_E, _H), jnp.float32),
    )(ga, gb, ea, w1c, w2, consts)


def _flux_body(en_ref, ei_ref, ea_ref, w1_ref, w2_ref, c_ref, fl_ref, eo_ref):
    en = en_ref[...]
    ei = ei_ref[...]
    h = jnp.dot(en + ei, w1_ref[...], preferred_element_type=jnp.float32)
    h = h + c_ref[0:1, :]
    h = jnp.dot(_gelu(h), w2_ref[...], preferred_element_type=jnp.float32)
    h = h + c_ref[1:2, :]
    fh = _ln(h, c_ref[2:3, :], c_ref[3:4, :])
    fl = en - ei + fh
    fl_ref[...] = fl
    eo_ref[...] = fl + ea_ref[...]


def _tc_flux(en, ei, ea, w1, w2, consts):
    eb = pl.BlockSpec((_BE, _H), lambda i: (i, 0))
    return pl.pallas_call(
        _flux_body,
        grid=(_GE,),
        in_specs=[eb, eb, eb, _full((_H, _H)), _full((_H, _H)), _full((8, _H))],
        out_specs=(eb, eb),
        out_shape=(
            jax.ShapeDtypeStruct((_E, _H), jnp.float32),
            jax.ShapeDtypeStruct((_E, _H), jnp.float32),
        ),
    )(en, ei, ea, w1, w2, consts)


def _upd_body(x_ref, aggp_ref, cntp_ref, w1a_ref, w1b_ref, w2_ref, c_ref, out_ref):
    x = x_ref[...]
    a = aggp_ref[0] + aggp_ref[1]
    cnt = jnp.maximum(cntp_ref[0, :, 0:1] + cntp_ref[1, :, 0:1], 1.0)
    agg = a / cnt
    h = jnp.dot(x, w1a_ref[...], preferred_element_type=jnp.float32) + jnp.dot(
        agg, w1b_ref[...], preferred_element_type=jnp.float32
    )
    h = h + c_ref[0:1, :]
    h = jnp.dot(_gelu(h), w2_ref[...], preferred_element_type=jnp.float32)
    h = h + c_ref[1:2, :]
    out_ref[...] = x + _ln(h, c_ref[2:3, :], c_ref[3:4, :])


def _tc_upd(x, aggp, cntp, w1a, w1b, w2, consts):
    return pl.pallas_call(
        _upd_body,
        out_shape=jax.ShapeDtypeStruct((_N, _H), jnp.float32),
    )(x, aggp, cntp, w1a, w1b, w2, consts)


def kernel(x, edge_index, edge_attr, pos, v, params):
    row = edge_index[0]
    col = edge_index[1]
    inv_perm = jnp.argsort(col * _N + row).astype(jnp.int32)

    row2 = row.reshape(_NW, _CPW, _CHUNK)
    col2 = col.reshape(_NW, _CPW, _CHUNK)
    perm2 = inv_perm.reshape(_NW, _CPW, _CHUNK)

    zeros_h = jnp.zeros((_N, _H), jnp.float32)

    cntp = _sc_count(row2, zeros_h)[:, :, :16]

    p = params
    for l in range(_L):
        w1 = p["msg_w1"][l]
        msg_c = jnp.stack(
            [p["msg_b2"][l], p["ln_g"][l], p["ln_b"][l]] + [p["ln_b"][l]] * 5
        )
        proj_c = jnp.stack([p["msg_b1"][l]] * 8)
        flux_c = jnp.stack(
            [p["flux_b1"][l], p["flux_b2"][l], p["ln_g"][l], p["ln_b"][l]] * 2
        )
        upd_c = jnp.stack(
            [p["upd_b1"][l], p["upd_b2"][l], p["ln_g"][l], p["ln_b"][l]] * 2
        )

        xa, xb = _tc_proj(x, w1[:_H], w1[_H : 2 * _H], proj_c)
        ga, gb = _sc_gather2(xa, xb, row2, col2)
        ea_new = _tc_msg(ga, gb, edge_attr, w1[2 * _H :], p["msg_w2"][l], msg_c)
        ea_inv = _sc_gather(ea_new, perm2)
        flux, edge_attr = _tc_flux(
            ea_new, ea_inv, edge_attr, p["flux_w1"][l], p["flux_w2"][l], flux_c
        )
        aggp = _sc_scatter_add(flux, row2, zeros_h, _H)
        x = _tc_upd(
            x,
            aggp,
            cntp,
            p["upd_w1"][l][:_H],
            p["upd_w1"][l][_H:],
            p["upd_w2"][l],
            upd_c,
        )
    return x, edge_attr


# speedup vs baseline: 3.5830x; 1.0777x over previous
import functools

import jax
import jax.numpy as jnp
from jax import lax
from jax.experimental import pallas as pl
from jax.experimental.pallas import tpu as pltpu
from jax.experimental.pallas import tpu_sc as plsc

_N = 10000
_E = 320000
_H = 128
_L = 3

_NC = 2
_NS = 16
_NW = _NC * _NS

_CHUNK = 80
_CPW = 125
_EPW = _CPW * _CHUNK
_RPT = 624

_BE = 4000
_GE = _E // _BE

@functools.cache
def _mesh():
    return plsc.VectorSubcoreMesh(
        core_axis_name="c", subcore_axis_name="s", num_cores=_NC, num_subcores=_NS
    )


def _pipe2(n, mk_in, mk_out, g0, g1, s0, s1, out_add=False):
    def step(j, carry):
        @pl.when(j % 2 == 0)
        def _():
            mk_in(j, 0, g0).wait()

            @pl.when(j + 1 < n)
            def _():
                @pl.when(j >= 1)
                def _():
                    mk_out(j - 1, 1, s1).wait()

                mk_in(j + 1, 1, g1).start()

            mk_out(j, 0, s0).start(add=out_add)

        @pl.when(j % 2 == 1)
        def _():
            mk_in(j, 1, g1).wait()

            @pl.when(j + 1 < n)
            def _():
                mk_out(j - 1, 0, s0).wait()
                mk_in(j + 1, 0, g0).start()

            mk_out(j, 1, s1).start(add=out_add)

        return carry

    mk_in(0, 0, g0).start()
    lax.fori_loop(0, n, step, 0)
    mk_out(n - 2, (n - 2) % 2, s0 if (n - 2) % 2 == 0 else s1).wait()
    mk_out(n - 1, (n - 1) % 2, s0 if (n - 1) % 2 == 0 else s1).wait()


def _sc_gather2(ta, tb, idxa3, idxb3):
    def body(ta_ref, tb_ref, ia_ref, ib_ref, oa_ref, ob_ref,
             ia_v, ib_v, rows_v, g0, g1, s0, s1):
        wid = lax.axis_index("s") * _NC + lax.axis_index("c")
        pltpu.sync_copy(ia_ref.at[wid], ia_v)
        pltpu.sync_copy(ib_ref.at[wid], ib_v)
        base = wid * _EPW

        def mk_in(j, b, sem):
            half = j // 2
            if b == 0:
                return pltpu.make_async_copy(
                    ta_ref.at[ia_v.at[half]], rows_v.at[b], sem
                )
            return pltpu.make_async_copy(
                tb_ref.at[ib_v.at[half]], rows_v.at[b], sem
            )

        def mk_out(j, b, sem):
            half = j // 2
            dst = oa_ref if b == 0 else ob_ref
            return pltpu.make_async_copy(
                rows_v.at[b], dst.at[pl.ds(base + half * _CHUNK, _CHUNK)], sem
            )

        _pipe2(2 * _CPW, mk_in, mk_out, g0, g1, s0, s1)

    return pl.kernel(
        body,
        out_type=(
            jax.ShapeDtypeStruct((_E, _H), jnp.float32),
            jax.ShapeDtypeStruct((_E, _H), jnp.float32),
        ),
        mesh=_mesh(),
        scratch_types=[
            pltpu.VMEM((_CPW, _CHUNK), jnp.int32),
            pltpu.VMEM((_CPW, _CHUNK), jnp.int32),
            pltpu.VMEM((2, _CHUNK, _H), jnp.float32),
            pltpu.SemaphoreType.DMA,
            pltpu.SemaphoreType.DMA,
            pltpu.SemaphoreType.DMA,
            pltpu.SemaphoreType.DMA,
        ],
    )(ta, tb, idxa3, idxb3)


def _sc_gather(table, idx3):
    def body(table_ref, idx_ref, out_ref, idx_v, rows_v, g0, g1, s0, s1):
        wid = lax.axis_index("s") * _NC + lax.axis_index("c")
        pltpu.sync_copy(idx_ref.at[wid], idx_v)
        base = wid * _EPW

        def mk_in(j, b, sem):
            return pltpu.make_async_copy(table_ref.at[idx_v.at[j]], rows_v.at[b], sem)

        def mk_out(j, b, sem):
            return pltpu.make_async_copy(
                rows_v.at[b], out_ref.at[pl.ds(base + j * _CHUNK, _CHUNK)], sem
            )

        _pipe2(_CPW, mk_in, mk_out, g0, g1, s0, s1)

    return pl.kernel(
        body,
        out_type=jax.ShapeDtypeStruct((_E, _H), jnp.float32),
        mesh=_mesh(),
        scratch_types=[
            pltpu.VMEM((_CPW, _CHUNK), jnp.int32),
            pltpu.VMEM((2, _CHUNK, _H), jnp.float32),
            pltpu.SemaphoreType.DMA,
            pltpu.SemaphoreType.DMA,
            pltpu.SemaphoreType.DMA,
            pltpu.SemaphoreType.DMA,
        ],
    )(table, idx3)


def _sc_scatter_add(vals, idx3, zeros, hc):
    def body(vals_ref, idx_ref, zeros_ref, out_ref, idx_v, rows_v, acc,
             g0, g1, s0, s1):
        cid = lax.axis_index("c")
        sid = lax.axis_index("s")
        wid = sid * _NC + cid
        @pl.when(sid < _NS - 1)
        def _():
            pltpu.sync_copy(
                zeros_ref.at[pl.ds(sid * _RPT, _RPT)],
                acc.at[pl.ds(sid * _RPT, _RPT)],
            )

        @pl.when(sid == _NS - 1)
        def _():
            lastn = _N - (_NS - 1) * _RPT
            pltpu.sync_copy(
                zeros_ref.at[pl.ds((_NS - 1) * _RPT, lastn)],
                acc.at[pl.ds((_NS - 1) * _RPT, lastn)],
            )

        pltpu.sync_copy(idx_ref.at[wid], idx_v)
        plsc.subcore_barrier()
        base = wid * _EPW

        def mk_in(j, b, sem):
            return pltpu.make_async_copy(
                vals_ref.at[pl.ds(base + j * _CHUNK, _CHUNK)], rows_v.at[b], sem
            )

        def mk_out(j, b, sem):
            return pltpu.make_async_copy(rows_v.at[b], acc.at[idx_v.at[j]], sem)

        _pipe2(_CPW, mk_in, mk_out, g0, g1, s0, s1, out_add=True)
        plsc.subcore_barrier()

        @pl.when(sid < _NS - 1)
        def _():
            pltpu.sync_copy(
                acc.at[pl.ds(sid * _RPT, _RPT)],
                out_ref.at[cid, pl.ds(sid * _RPT, _RPT)],
            )

        @pl.when(sid == _NS - 1)
        def _():
            lastn = _N - (_NS - 1) * _RPT
            pltpu.sync_copy(
                acc.at[pl.ds((_NS - 1) * _RPT, lastn)],
                out_ref.at[cid, pl.ds((_NS - 1) * _RPT, lastn)],
            )

    return pl.kernel(
        body,
        out_type=jax.ShapeDtypeStruct((_NC, _N, hc), jnp.float32),
        mesh=_mesh(),
        scratch_types=[
            pltpu.VMEM((_CPW, _CHUNK), jnp.int32),
            pltpu.VMEM((2, _CHUNK, hc), jnp.float32),
            pltpu.VMEM_SHARED((_N, hc), jnp.float32),
            pltpu.SemaphoreType.DMA,
            pltpu.SemaphoreType.DMA,
            pltpu.SemaphoreType.DMA,
            pltpu.SemaphoreType.DMA,
        ],
    )(vals, idx3, zeros)


def _sc_count(idx3, zeros):
    def body(idx_ref, zeros_ref, out_ref, idx_v, ones_v, acc, sem):
        cid = lax.axis_index("c")
        sid = lax.axis_index("s")
        wid = sid * _NC + cid

        @pl.when(sid < _NS - 1)
        def _():
            pltpu.sync_copy(
                zeros_ref.at[pl.ds(sid * _RPT, _RPT)],
                acc.at[pl.ds(sid * _RPT, _RPT)],
            )

        @pl.when(sid == _NS - 1)
        def _():
            lastn = _N - (_NS - 1) * _RPT
            pltpu.sync_copy(
                zeros_ref.at[pl.ds((_NS - 1) * _RPT, lastn)],
                acc.at[pl.ds((_NS - 1) * _RPT, lastn)],
            )

        pltpu.sync_copy(idx_ref.at[wid], idx_v)

        def fill(i, c):
            def fill2(k, c2):
                ones_v[i, pl.ds(k * 16, 16)] = jnp.ones((16,), jnp.float32)
                return c2

            return lax.fori_loop(0, _H // 16, fill2, c)

        lax.fori_loop(0, _CHUNK, fill, 0)
        plsc.subcore_barrier()

        def mk(j):
            return pltpu.make_async_copy(ones_v, acc.at[idx_v.at[j]], sem)

        def step(j, carry):
            mk(j).start(add=True)

            @pl.when(j >= 4)
            def _():
                mk(j - 4).wait()

            return carry

        lax.fori_loop(0, _CPW, step, 0)
        for j in range(4):
            mk(_CPW - 4 + j).wait()
        plsc.subcore_barrier()

        @pl.when(sid < _NS - 1)
        def _():
            pltpu.sync_copy(
                acc.at[pl.ds(sid * _RPT, _RPT)],
                out_ref.at[cid, pl.ds(sid * _RPT, _RPT)],
            )

        @pl.when(sid == _NS - 1)
        def _():
            lastn = _N - (_NS - 1) * _RPT
            pltpu.sync_copy(
                acc.at[pl.ds((_NS - 1) * _RPT, lastn)],
                out_ref.at[cid, pl.ds((_NS - 1) * _RPT, lastn)],
            )

    return pl.kernel(
        body,
        out_type=jax.ShapeDtypeStruct((_NC, _N, _H), jnp.float32),
        mesh=_mesh(),
        scratch_types=[
            pltpu.VMEM((_CPW, _CHUNK), jnp.int32),
            pltpu.VMEM((_CHUNK, _H), jnp.float32),
            pltpu.VMEM_SHARED((_N, _H), jnp.float32),
            pltpu.SemaphoreType.DMA,
        ],
    )(idx3, zeros)


def _gelu(x):
    return 0.5 * x * (1.0 + lax.erf(x * 0.7071067811865476))


def _ln(h, g, b):
    m = jnp.mean(h, axis=-1, keepdims=True)
    d = h - m
    v = jnp.mean(d * d, axis=-1, keepdims=True)
    return d * lax.rsqrt(v + 1e-5) * g + b


def _full(shape):
    return pl.BlockSpec(shape, lambda i: (0,) * len(shape))


def _proj_body(x_ref, w1a_ref, w1b_ref, c_ref, xa_ref, xb_ref):
    x = x_ref[...]
    xa_ref[...] = (
        jnp.dot(x, w1a_ref[...], preferred_element_type=jnp.float32) + c_ref[0:1, :]
    )
    xb_ref[...] = jnp.dot(x, w1b_ref[...], preferred_element_type=jnp.float32)


def _tc_proj(x, w1a, w1b, consts):
    return pl.pallas_call(
        _proj_body,
        out_shape=(
            jax.ShapeDtypeStruct((_N, _H), jnp.float32),
            jax.ShapeDtypeStruct((_N, _H), jnp.float32),
        ),
    )(x, w1a, w1b, consts)


def _msg_body(ga_ref, gb_ref, ea_ref, w1c_ref, w2_ref, c_ref, out_ref):
    h = (
        ga_ref[...]
        + gb_ref[...]
        + jnp.dot(ea_ref[...], w1c_ref[...], preferred_element_type=jnp.float32)
    )
    h = jnp.dot(_gelu(h), w2_ref[...], preferred_element_type=jnp.float32)
    h = h + c_ref[0:1, :]
    out_ref[...] = _ln(h, c_ref[1:2, :], c_ref[2:3, :])


def _tc_msg(ga, gb, ea, w1c, w2, consts):
    eb = pl.BlockSpec((_BE, _H), lambda i: (i, 0))
    return pl.pallas_call(
        _msg_body,
        grid=(_GE,),
        in_specs=[eb, eb, eb, _full((_H, _H)), _full((_H, _H)), _full((8, _H))],
        out_specs=eb,
        out_shape=jax.ShapeDtypeStruct((_E, _H), jnp.float32),
    )(ga, gb, ea, w1c, w2, consts)


def _flux_body(en_ref, ei_ref, ea_ref, w1_ref, w2_ref, c_ref, fl_ref, eo_ref):
    en = en_ref[...]
    ei = ei_ref[...]
    h = jnp.dot(en + ei, w1_ref[...], preferred_element_type=jnp.float32)
    h = h + c_ref[0:1, :]
    h = jnp.dot(_gelu(h), w2_ref[...], preferred_element_type=jnp.float32)
    h = h + c_ref[1:2, :]
    fh = _ln(h, c_ref[2:3, :], c_ref[3:4, :])
    fl = en - ei + fh
    fl_ref[...] = fl
    eo_ref[...] = fl + ea_ref[...]


def _tc_flux(en, ei, ea, w1, w2, consts):
    eb = pl.BlockSpec((_BE, _H), lambda i: (i, 0))
    return pl.pallas_call(
        _flux_body,
        grid=(_GE,),
        in_specs=[eb, eb, eb, _full((_H, _H)), _full((_H, _H)), _full((8, _H))],
        out_specs=(eb, eb),
        out_shape=(
            jax.ShapeDtypeStruct((_E, _H), jnp.float32),
            jax.ShapeDtypeStruct((_E, _H), jnp.float32),
        ),
    )(en, ei, ea, w1, w2, consts)


def _upd_body(x_ref, aggp_ref, cntp_ref, w1a_ref, w1b_ref, w2_ref, c_ref, out_ref):
    x = x_ref[...]
    a = aggp_ref[0] + aggp_ref[1]
    cnt = jnp.maximum(cntp_ref[0, :, 0:1] + cntp_ref[1, :, 0:1], 1.0)
    agg = a / cnt
    h = jnp.dot(x, w1a_ref[...], preferred_element_type=jnp.float32) + jnp.dot(
        agg, w1b_ref[...], preferred_element_type=jnp.float32
    )
    h = h + c_ref[0:1, :]
    h = jnp.dot(_gelu(h), w2_ref[...], preferred_element_type=jnp.float32)
    h = h + c_ref[1:2, :]
    out_ref[...] = x + _ln(h, c_ref[2:3, :], c_ref[3:4, :])


def _tc_upd(x, aggp, cntp, w1a, w1b, w2, consts):
    return pl.pallas_call(
        _upd_body,
        out_shape=jax.ShapeDtypeStruct((_N, _H), jnp.float32),
    )(x, aggp, cntp, w1a, w1b, w2, consts)


def kernel(x, edge_index, edge_attr, pos, v, params):
    row = edge_index[0]
    col = edge_index[1]
    inv_perm = jnp.argsort(col * _N + row).astype(jnp.int32)

    row2 = row.reshape(_NW, _CPW, _CHUNK)
    col2 = col.reshape(_NW, _CPW, _CHUNK)
    perm2 = inv_perm.reshape(_NW, _CPW, _CHUNK)

    zeros_h = jnp.zeros((_N, _H), jnp.float32)

    cntp = _sc_count(row2, zeros_h)[:, :, :16]

    p = params
    for l in range(_L):
        w1 = p["msg_w1"][l]
        msg_c = jnp.stack(
            [p["msg_b2"][l], p["ln_g"][l], p["ln_b"][l]] + [p["ln_b"][l]] * 5
        )
        proj_c = jnp.stack([p["msg_b1"][l]] * 8)
        flux_c = jnp.stack(
            [p["flux_b1"][l], p["flux_b2"][l], p["ln_g"][l], p["ln_b"][l]] * 2
        )
        upd_c = jnp.stack(
            [p["upd_b1"][l], p["upd_b2"][l], p["ln_g"][l], p["ln_b"][l]] * 2
        )

        xa, xb = _tc_proj(x, w1[:_H], w1[_H : 2 * _H], proj_c)
        ga, gb = _sc_gather2(xa, xb, row2, col2)
        ea_new = _tc_msg(ga, gb, edge_attr, w1[2 * _H :], p["msg_w2"][l], msg_c)
        ea_inv = _sc_gather(ea_new, perm2)
        flux, edge_attr = _tc_flux(
            ea_new, ea_inv, edge_attr, p["flux_w1"][l], p["flux_w2"][l], flux_c
        )
        aggp = _sc_scatter_add(flux, row2, zeros_h, _H)
        x = _tc_upd(
            x,
            aggp,
            cntp,
            p["upd_w1"][l][:_H],
            p["upd_w1"][l][_H:],
            p["upd_w2"][l],
            upd_c,
        )
    return x, edge_attr


# speedup vs baseline: 3.6452x; 1.0174x over previous
import functools

import jax
import jax.numpy as jnp
from jax import lax
from jax.experimental import pallas as pl
from jax.experimental.pallas import tpu as pltpu
from jax.experimental.pallas import tpu_sc as plsc

_N = 10000
_E = 320000
_H = 128
_L = 3

_NC = 2
_NS = 16
_NW = _NC * _NS

_CHUNK = 80
_CPW = 125
_EPW = _CPW * _CHUNK
_RPT = 624

_BE = 8000
_GE = _E // _BE

@functools.cache
def _mesh():
    return plsc.VectorSubcoreMesh(
        core_axis_name="c", subcore_axis_name="s", num_cores=_NC, num_subcores=_NS
    )


def _pipe2(n, mk_in, mk_out, g0, g1, s0, s1, out_add=False):
    def step(j, carry):
        @pl.when(j % 2 == 0)
        def _():
            mk_in(j, 0, g0).wait()

            @pl.when(j + 1 < n)
            def _():
                @pl.when(j >= 1)
                def _():
                    mk_out(j - 1, 1, s1).wait()

                mk_in(j + 1, 1, g1).start()

            mk_out(j, 0, s0).start(add=out_add)

        @pl.when(j % 2 == 1)
        def _():
            mk_in(j, 1, g1).wait()

            @pl.when(j + 1 < n)
            def _():
                mk_out(j - 1, 0, s0).wait()
                mk_in(j + 1, 0, g0).start()

            mk_out(j, 1, s1).start(add=out_add)

        return carry

    mk_in(0, 0, g0).start()
    lax.fori_loop(0, n, step, 0)
    mk_out(n - 2, (n - 2) % 2, s0 if (n - 2) % 2 == 0 else s1).wait()
    mk_out(n - 1, (n - 1) % 2, s0 if (n - 1) % 2 == 0 else s1).wait()


def _sc_gather2(ta, tb, idxa3, idxb3):
    def body(ta_ref, tb_ref, ia_ref, ib_ref, oa_ref, ob_ref,
             ia_v, ib_v, rows_v, g0, g1, s0, s1):
        wid = lax.axis_index("s") * _NC + lax.axis_index("c")
        pltpu.sync_copy(ia_ref.at[wid], ia_v)
        pltpu.sync_copy(ib_ref.at[wid], ib_v)
        base = wid * _EPW

        def mk_in(j, b, sem):
            half = j // 2
            if b == 0:
                return pltpu.make_async_copy(
                    ta_ref.at[ia_v.at[half]], rows_v.at[b], sem
                )
            return pltpu.make_async_copy(
                tb_ref.at[ib_v.at[half]], rows_v.at[b], sem
            )

        def mk_out(j, b, sem):
            half = j // 2
            dst = oa_ref if b == 0 else ob_ref
            return pltpu.make_async_copy(
                rows_v.at[b], dst.at[pl.ds(base + half * _CHUNK, _CHUNK)], sem
            )

        _pipe2(2 * _CPW, mk_in, mk_out, g0, g1, s0, s1)

    return pl.kernel(
        body,
        out_type=(
            jax.ShapeDtypeStruct((_E, _H), jnp.float32),
            jax.ShapeDtypeStruct((_E, _H), jnp.float32),
        ),
        mesh=_mesh(),
        scratch_types=[
            pltpu.VMEM((_CPW, _CHUNK), jnp.int32),
            pltpu.VMEM((_CPW, _CHUNK), jnp.int32),
            pltpu.VMEM((2, _CHUNK, _H), jnp.float32),
            pltpu.SemaphoreType.DMA,
            pltpu.SemaphoreType.DMA,
            pltpu.SemaphoreType.DMA,
            pltpu.SemaphoreType.DMA,
        ],
    )(ta, tb, idxa3, idxb3)


def _sc_gather(table, idx3):
    def body(table_ref, idx_ref, out_ref, idx_v, rows_v, g0, g1, s0, s1):
        wid = lax.axis_index("s") * _NC + lax.axis_index("c")
        pltpu.sync_copy(idx_ref.at[wid], idx_v)
        base = wid * _EPW

        def mk_in(j, b, sem):
            return pltpu.make_async_copy(table_ref.at[idx_v.at[j]], rows_v.at[b], sem)

        def mk_out(j, b, sem):
            return pltpu.make_async_copy(
                rows_v.at[b], out_ref.at[pl.ds(base + j * _CHUNK, _CHUNK)], sem
            )

        _pipe2(_CPW, mk_in, mk_out, g0, g1, s0, s1)

    return pl.kernel(
        body,
        out_type=jax.ShapeDtypeStruct((_E, _H), jnp.float32),
        mesh=_mesh(),
        scratch_types=[
            pltpu.VMEM((_CPW, _CHUNK), jnp.int32),
            pltpu.VMEM((2, _CHUNK, _H), jnp.float32),
            pltpu.SemaphoreType.DMA,
            pltpu.SemaphoreType.DMA,
            pltpu.SemaphoreType.DMA,
            pltpu.SemaphoreType.DMA,
        ],
    )(table, idx3)


def _sc_scatter_add(vals, idx3, zeros, hc):
    def body(vals_ref, idx_ref, zeros_ref, out_ref, idx_v, rows_v, acc,
             g0, g1, s0, s1):
        cid = lax.axis_index("c")
        sid = lax.axis_index("s")
        wid = sid * _NC + cid
        @pl.when(sid < _NS - 1)
        def _():
            pltpu.sync_copy(
                zeros_ref.at[pl.ds(sid * _RPT, _RPT)],
                acc.at[pl.ds(sid * _RPT, _RPT)],
            )

        @pl.when(sid == _NS - 1)
        def _():
            lastn = _N - (_NS - 1) * _RPT
            pltpu.sync_copy(
                zeros_ref.at[pl.ds((_NS - 1) * _RPT, lastn)],
                acc.at[pl.ds((_NS - 1) * _RPT, lastn)],
            )

        pltpu.sync_copy(idx_ref.at[wid], idx_v)
        plsc.subcore_barrier()
        base = wid * _EPW

        def mk_in(j, b, sem):
            return pltpu.make_async_copy(
                vals_ref.at[pl.ds(base + j * _CHUNK, _CHUNK)], rows_v.at[b], sem
            )

        def mk_out(j, b, sem):
            return pltpu.make_async_copy(rows_v.at[b], acc.at[idx_v.at[j]], sem)

        _pipe2(_CPW, mk_in, mk_out, g0, g1, s0, s1, out_add=True)
        plsc.subcore_barrier()

        @pl.when(sid < _NS - 1)
        def _():
            pltpu.sync_copy(
                acc.at[pl.ds(sid * _RPT, _RPT)],
                out_ref.at[cid, pl.ds(sid * _RPT, _RPT)],
            )

        @pl.when(sid == _NS - 1)
        def _():
            lastn = _N - (_NS - 1) * _RPT
            pltpu.sync_copy(
                acc.at[pl.ds((_NS - 1) * _RPT, lastn)],
                out_ref.at[cid, pl.ds((_NS - 1) * _RPT, lastn)],
            )

    return pl.kernel(
        body,
        out_type=jax.ShapeDtypeStruct((_NC, _N, hc), jnp.float32),
        mesh=_mesh(),
        scratch_types=[
            pltpu.VMEM((_CPW, _CHUNK), jnp.int32),
            pltpu.VMEM((2, _CHUNK, hc), jnp.float32),
            pltpu.VMEM_SHARED((_N, hc), jnp.float32),
            pltpu.SemaphoreType.DMA,
            pltpu.SemaphoreType.DMA,
            pltpu.SemaphoreType.DMA,
            pltpu.SemaphoreType.DMA,
        ],
    )(vals, idx3, zeros)


def _sc_count(idx3, zeros):
    def body(idx_ref, zeros_ref, out_ref, idx_v, ones_v, acc, sem):
        cid = lax.axis_index("c")
        sid = lax.axis_index("s")
        wid = sid * _NC + cid

        @pl.when(sid < _NS - 1)
        def _():
            pltpu.sync_copy(
                zeros_ref.at[pl.ds(sid * _RPT, _RPT)],
                acc.at[pl.ds(sid * _RPT, _RPT)],
            )

        @pl.when(sid == _NS - 1)
        def _():
            lastn = _N - (_NS - 1) * _RPT
            pltpu.sync_copy(
                zeros_ref.at[pl.ds((_NS - 1) * _RPT, lastn)],
                acc.at[pl.ds((_NS - 1) * _RPT, lastn)],
            )

        pltpu.sync_copy(idx_ref.at[wid], idx_v)

        def fill(i, c):
            def fill2(k, c2):
                ones_v[i, pl.ds(k * 16, 16)] = jnp.ones((16,), jnp.float32)
                return c2

            return lax.fori_loop(0, _H // 16, fill2, c)

        lax.fori_loop(0, _CHUNK, fill, 0)
        plsc.subcore_barrier()

        def mk(j):
            return pltpu.make_async_copy(ones_v, acc.at[idx_v.at[j]], sem)

        def step(j, carry):
            mk(j).start(add=True)

            @pl.when(j >= 4)
            def _():
                mk(j - 4).wait()

            return carry

        lax.fori_loop(0, _CPW, step, 0)
        for j in range(4):
            mk(_CPW - 4 + j).wait()
        plsc.subcore_barrier()

        @pl.when(sid < _NS - 1)
        def _():
            pltpu.sync_copy(
                acc.at[pl.ds(sid * _RPT, _RPT)],
                out_ref.at[cid, pl.ds(sid * _RPT, _RPT)],
            )

        @pl.when(sid == _NS - 1)
        def _():
            lastn = _N - (_NS - 1) * _RPT
            pltpu.sync_copy(
                acc.at[pl.ds((_NS - 1) * _RPT, lastn)],
                out_ref.at[cid, pl.ds((_NS - 1) * _RPT, lastn)],
            )

    return pl.kernel(
        body,
        out_type=jax.ShapeDtypeStruct((_NC, _N, _H), jnp.float32),
        mesh=_mesh(),
        scratch_types=[
            pltpu.VMEM((_CPW, _CHUNK), jnp.int32),
            pltpu.VMEM((_CHUNK, _H), jnp.float32),
            pltpu.VMEM_SHARED((_N, _H), jnp.float32),
            pltpu.SemaphoreType.DMA,
        ],
    )(idx3, zeros)


def _gelu(x):
    return 0.5 * x * (1.0 + lax.erf(x * 0.7071067811865476))


def _ln(h, g, b):
    m = jnp.mean(h, axis=-1, keepdims=True)
    d = h - m
    v = jnp.mean(d * d, axis=-1, keepdims=True)
    return d * lax.rsqrt(v + 1e-5) * g + b


def _full(shape):
    return pl.BlockSpec(shape, lambda i: (0,) * len(shape))


def _proj_body(x_ref, w1a_ref, w1b_ref, c_ref, xa_ref, xb_ref):
    x = x_ref[...]
    xa_ref[...] = (
        jnp.dot(x, w1a_ref[...], preferred_element_type=jnp.float32) + c_ref[0:1, :]
    )
    xb_ref[...] = jnp.dot(x, w1b_ref[...], preferred_element_type=jnp.float32)


def _tc_proj(x, w1a, w1b, consts):
    return pl.pallas_call(
        _proj_body,
        out_shape=(
            jax.ShapeDtypeStruct((_N, _H), jnp.float32),
            jax.ShapeDtypeStruct((_N, _H), jnp.float32),
        ),
    )(x, w1a, w1b, consts)


def _msg_body(ga_ref, gb_ref, ea_ref, w1c_ref, w2_ref, c_ref, out_ref):
    h = (
        ga_ref[...]
        + gb_ref[...]
        + jnp.dot(ea_ref[...], w1c_ref[...], preferred_element_type=jnp.float32)
    )
    h = jnp.dot(_gelu(h), w2_ref[...], preferred_element_type=jnp.float32)
    h = h + c_ref[0:1, :]
    out_ref[...] = _ln(h, c_ref[1:2, :], c_ref[2:3, :])


def _tc_msg(ga, gb, ea, w1c, w2, consts):
    eb = pl.BlockSpec((_BE, _H), lambda i: (i, 0))
    return pl.pallas_call(
        _msg_body,
        grid=(_GE,),
        in_specs=[eb, eb, eb, _full((_H, _H)), _full((_H, _H)), _full((8, _H))],
        out_specs=eb,
        out_shape=jax.ShapeDtypeStruct((_E, _H), jnp.float32),
    )(ga, gb, ea, w1c, w2, consts)


def _flux_body(en_ref, ei_ref, ea_ref, w1_ref, w2_ref, c_ref, fl_ref, eo_ref):
    en = en_ref[...]
    ei = ei_ref[...]
    h = jnp.dot(en + ei, w1_ref[...], preferred_element_type=jnp.float32)
    h = h + c_ref[0:1, :]
    h = jnp.dot(_gelu(h), w2_ref[...], preferred_element_type=jnp.float32)
    h = h + c_ref[1:2, :]
    fh = _ln(h, c_ref[2:3, :], c_ref[3:4, :])
    fl = en - ei + fh
    fl_ref[...] = fl
    eo_ref[...] = fl + ea_ref[...]


def _tc_flux(en, ei, ea, w1, w2, consts):
    eb = pl.BlockSpec((_BE, _H), lambda i: (i, 0))
    return pl.pallas_call(
        _flux_body,
        grid=(_GE,),
        in_specs=[eb, eb, eb, _full((_H, _H)), _full((_H, _H)), _full((8, _H))],
        out_specs=(eb, eb),
        out_shape=(
            jax.ShapeDtypeStruct((_E, _H), jnp.float32),
            jax.ShapeDtypeStruct((_E, _H), jnp.float32),
        ),
    )(en, ei, ea, w1, w2, consts)


def _upd_body(x_ref, aggp_ref, cntp_ref, w1a_ref, w1b_ref, w2_ref, c_ref, out_ref):
    x = x_ref[...]
    a = aggp_ref[0] + aggp_ref[1]
    cnt = jnp.maximum(cntp_ref[0, :, 0:1] + cntp_ref[1, :, 0:1], 1.0)
    agg = a / cnt
    h = jnp.dot(x, w1a_ref[...], preferred_element_type=jnp.float32) + jnp.dot(
        agg, w1b_ref[...], preferred_element_type=jnp.float32
    )
    h = h + c_ref[0:1, :]
    h = jnp.dot(_gelu(h), w2_ref[...], preferred_element_type=jnp.float32)
    h = h + c_ref[1:2, :]
    out_ref[...] = x + _ln(h, c_ref[2:3, :], c_ref[3:4, :])


def _tc_upd(x, aggp, cntp, w1a, w1b, w2, consts):
    return pl.pallas_call(
        _upd_body,
        out_shape=jax.ShapeDtypeStruct((_N, _H), jnp.float32),
    )(x, aggp, cntp, w1a, w1b, w2, consts)


def kernel(x, edge_index, edge_attr, pos, v, params):
    row = edge_index[0]
    col = edge_index[1]
    inv_perm = jnp.argsort(col * _N + row).astype(jnp.int32)

    row2 = row.reshape(_NW, _CPW, _CHUNK)
    col2 = col.reshape(_NW, _CPW, _CHUNK)
    perm2 = inv_perm.reshape(_NW, _CPW, _CHUNK)

    zeros_h = jnp.zeros((_N, _H), jnp.float32)

    cntp = _sc_count(row2, zeros_h)[:, :, :16]

    p = params
    for l in range(_L):
        w1 = p["msg_w1"][l]
        msg_c = jnp.stack(
            [p["msg_b2"][l], p["ln_g"][l], p["ln_b"][l]] + [p["ln_b"][l]] * 5
        )
        proj_c = jnp.stack([p["msg_b1"][l]] * 8)
        flux_c = jnp.stack(
            [p["flux_b1"][l], p["flux_b2"][l], p["ln_g"][l], p["ln_b"][l]] * 2
        )
        upd_c = jnp.stack(
            [p["upd_b1"][l], p["upd_b2"][l], p["ln_g"][l], p["ln_b"][l]] * 2
        )

        xa, xb = _tc_proj(x, w1[:_H], w1[_H : 2 * _H], proj_c)
        ga, gb = _sc_gather2(xa, xb, row2, col2)
        ea_new = _tc_msg(ga, gb, edge_attr, w1[2 * _H :], p["msg_w2"][l], msg_c)
        ea_inv = _sc_gather(ea_new, perm2)
        flux, edge_attr = _tc_flux(
            ea_new, ea_inv, edge_attr, p["flux_w1"][l], p["flux_w2"][l], flux_c
        )
        aggp = _sc_scatter_add(flux, row2, zeros_h, _H)
        x = _tc_upd(
            x,
            aggp,
            cntp,
            p["upd_w1"][l][:_H],
            p["upd_w1"][l][_H:],
            p["upd_w2"][l],
            upd_c,
        )
    return x, edge_attr
